# baseline (speedup 1.0000x reference)
import jax
import jax.numpy as jnp
from jax import lax
from jax.experimental import pallas as pl
from jax.experimental.pallas import tpu as pltpu
from jax.experimental.pallas import tpu_sc as plsc

D = 128
L = 16
NC = 2
NS = 16
NW = NC * NS
B = 80
NG = B // L


def _ip_body(xu, xm, eidx, out, idxu_all, idxm_all,
             u0, m0, u1, m1, ob0, ob1, tmat,
             su0, sm0, su1, sm1, so0, so1):
    wid = lax.axis_index("s") * NC + lax.axis_index("c")
    n_edges = out.shape[0]
    epw = n_edges // NW
    chunks = epw // B
    base = wid * epw

    pltpu.sync_copy(eidx.at[pl.ds(base, epw)], idxu_all)
    pltpu.sync_copy(eidx.at[pl.ds(n_edges + base, epw)], idxm_all)

    def issue(c, ub, mb, su, sm):
        o = c * B
        pltpu.async_copy(xu.at[idxu_all.at[pl.ds(o, B)]], ub, su)
        pltpu.async_copy(xm.at[idxm_all.at[pl.ds(o, B)]], mb, sm)

    def wait(ub, mb, su, sm):
        pltpu.make_async_copy(xu.at[idxu_all.at[pl.ds(0, B)]], ub, su).wait()
        pltpu.make_async_copy(xm.at[idxm_all.at[pl.ds(0, B)]], mb, sm).wait()

    lane = lax.iota(jnp.int32, L)

    def rowsum(gi, ob):
        rows = [tmat[r, pl.ds(gi * 17, L)] for r in range(L)]
        while len(rows) > 1:
            rows = [rows[i] + rows[i + 1] for i in range(0, len(rows), 2)]
        ob[pl.ds(gi * L, L)] = rows[0]

    def compute(c, ub, mb, ob, so):
        @pl.when(c >= 2)
        def _():
            pltpu.make_async_copy(ob, out.at[pl.ds(0, B)], so).wait()

        @plsc.parallel_loop(0, NG)
        def group_body(g):
            e0 = g * L
            col0 = g * 17
            for t in range(L):
                e = e0 + t
                a0 = ub[e, pl.ds(0, L)] * mb[e, pl.ds(0, L)]
                a1 = ub[e, pl.ds(L, L)] * mb[e, pl.ds(L, L)]
                for k in range(2, D // L, 2):
                    a0 = a0 + ub[e, pl.ds(k * L, L)] * mb[e, pl.ds(k * L, L)]
                    a1 = a1 + ub[e, pl.ds((k + 1) * L, L)] * mb[e, pl.ds((k + 1) * L, L)]
                plsc.store_scatter(tmat,
                                   [lane, jnp.full((L,), t, jnp.int32) + col0],
                                   a0 + a1)

        for g in range(NG):
            rowsum(g, ob)
        pltpu.async_copy(ob, out.at[pl.ds(base + c * B, B)], so)

    issue(0, u0, m0, su0, sm0)

    def pair_body(j, carry):
        c = 2 * j
        issue(c + 1, u1, m1, su1, sm1)
        wait(u0, m0, su0, sm0)
        compute(c, u0, m0, ob0, so0)
        issue(c + 2, u0, m0, su0, sm0)
        wait(u1, m1, su1, sm1)
        compute(c + 1, u1, m1, ob1, so1)
        return carry

    lax.fori_loop(0, (chunks - 1) // 2, pair_body, 0)
    wait(u0, m0, su0, sm0)
    compute(chunks - 1, u0, m0, ob0, so0)

    pltpu.make_async_copy(ob0, out.at[pl.ds(0, B)], so0).wait()
    pltpu.make_async_copy(ob1, out.at[pl.ds(0, B)], so1).wait()


def kernel(x_user, x_movie, edge_label_index):
    n_edges = edge_label_index.shape[1]
    epw = n_edges // NW
    mesh = plsc.VectorSubcoreMesh(core_axis_name="c", subcore_axis_name="s")
    f = pl.kernel(
        _ip_body,
        out_type=jax.ShapeDtypeStruct((n_edges,), jnp.float32),
        mesh=mesh,
        compiler_params=pltpu.CompilerParams(needs_layout_passes=False),
        scratch_types=[
            pltpu.VMEM((epw,), jnp.int32),
            pltpu.VMEM((epw,), jnp.int32),
            pltpu.VMEM((B, D), jnp.float32),
            pltpu.VMEM((B, D), jnp.float32),
            pltpu.VMEM((B, D), jnp.float32),
            pltpu.VMEM((B, D), jnp.float32),
            pltpu.VMEM((B,), jnp.float32),
            pltpu.VMEM((B,), jnp.float32),
            pltpu.VMEM((L, 85), jnp.float32),
            pltpu.SemaphoreType.DMA,
            pltpu.SemaphoreType.DMA,
            pltpu.SemaphoreType.DMA,
            pltpu.SemaphoreType.DMA,
            pltpu.SemaphoreType.DMA,
            pltpu.SemaphoreType.DMA,
        ],
    )
    return f(x_user, x_movie, edge_label_index.reshape(2 * n_edges))


# speedup vs baseline: 1.0545x; 1.0545x over previous
import jax
import jax.numpy as jnp
from jax import lax
from jax.experimental import pallas as pl
from jax.experimental.pallas import tpu as pltpu
from jax.experimental.pallas import tpu_sc as plsc

D = 128
L = 16
NC = 2
NS = 16
NW = NC * NS
B = 80
NG = B // L


def _ip_body(xu, xm, eidx, out, idxu_all, idxm_all,
             u0, m0, u1, m1, ob0, ob1, tmat,
             su0, sm0, su1, sm1, so0, so1):
    wid = lax.axis_index("s") * NC + lax.axis_index("c")
    n_edges = out.shape[0]
    epw = n_edges // NW
    chunks = epw // B
    base = wid * epw

    pltpu.sync_copy(eidx.at[pl.ds(base, epw)], idxu_all)
    pltpu.sync_copy(eidx.at[pl.ds(n_edges + base, epw)], idxm_all)

    def issue(c, ub, mb, su, sm):
        o = c * B
        pltpu.async_copy(xu.at[idxu_all.at[pl.ds(o, B)]], ub, su)
        pltpu.async_copy(xm.at[idxm_all.at[pl.ds(o, B)]], mb, sm)

    def wait(ub, mb, su, sm):
        pltpu.make_async_copy(xu.at[idxu_all.at[pl.ds(0, B)]], ub, su).wait()
        pltpu.make_async_copy(xm.at[idxm_all.at[pl.ds(0, B)]], mb, sm).wait()

    lane = lax.iota(jnp.int32, L)

    def rowsum(gi, ob):
        rows = [tmat[r, pl.ds(gi * 17, L)] for r in range(L)]
        while len(rows) > 1:
            rows = [rows[i] + rows[i + 1] for i in range(0, len(rows), 2)]
        ob[pl.ds(gi * L, L)] = rows[0]

    def compute(c, ub, mb, ob, so):
        @pl.when(c >= 2)
        def _():
            pltpu.make_async_copy(ob, out.at[pl.ds(0, B)], so).wait()

        def group_body(g, c2):
            e0 = g * L
            ub_g = ub.at[pl.ds(e0, L)]
            mb_g = mb.at[pl.ds(e0, L)]
            colv = lane * 0 + g * 17
            for t in range(L):
                a0 = ub_g[t, pl.ds(0, L)] * mb_g[t, pl.ds(0, L)]
                a1 = ub_g[t, pl.ds(L, L)] * mb_g[t, pl.ds(L, L)]
                for k in range(2, D // L, 2):
                    a0 = a0 + ub_g[t, pl.ds(k * L, L)] * mb_g[t, pl.ds(k * L, L)]
                    a1 = a1 + ub_g[t, pl.ds((k + 1) * L, L)] * mb_g[t, pl.ds((k + 1) * L, L)]
                plsc.store_scatter(tmat, [lane, colv + t], a0 + a1)
            return c2

        lax.fori_loop(0, NG, group_body, 0)
        for g in range(NG):
            rowsum(g, ob)
        pltpu.async_copy(ob, out.at[pl.ds(base + c * B, B)], so)

    issue(0, u0, m0, su0, sm0)

    def pair_body(j, carry):
        c = 2 * j
        issue(c + 1, u1, m1, su1, sm1)
        wait(u0, m0, su0, sm0)
        compute(c, u0, m0, ob0, so0)
        issue(c + 2, u0, m0, su0, sm0)
        wait(u1, m1, su1, sm1)
        compute(c + 1, u1, m1, ob1, so1)
        return carry

    lax.fori_loop(0, (chunks - 1) // 2, pair_body, 0)
    wait(u0, m0, su0, sm0)
    compute(chunks - 1, u0, m0, ob0, so0)

    pltpu.make_async_copy(ob0, out.at[pl.ds(0, B)], so0).wait()
    pltpu.make_async_copy(ob1, out.at[pl.ds(0, B)], so1).wait()


def kernel(x_user, x_movie, edge_label_index):
    n_edges = edge_label_index.shape[1]
    epw = n_edges // NW
    mesh = plsc.VectorSubcoreMesh(core_axis_name="c", subcore_axis_name="s")
    f = pl.kernel(
        _ip_body,
        out_type=jax.ShapeDtypeStruct((n_edges,), jnp.float32),
        mesh=mesh,
        compiler_params=pltpu.CompilerParams(needs_layout_passes=False),
        scratch_types=[
            pltpu.VMEM((epw,), jnp.int32),
            pltpu.VMEM((epw,), jnp.int32),
            pltpu.VMEM((B, D), jnp.float32),
            pltpu.VMEM((B, D), jnp.float32),
            pltpu.VMEM((B, D), jnp.float32),
            pltpu.VMEM((B, D), jnp.float32),
            pltpu.VMEM((B,), jnp.float32),
            pltpu.VMEM((B,), jnp.float32),
            pltpu.VMEM((L, 85), jnp.float32),
            pltpu.SemaphoreType.DMA,
            pltpu.SemaphoreType.DMA,
            pltpu.SemaphoreType.DMA,
            pltpu.SemaphoreType.DMA,
            pltpu.SemaphoreType.DMA,
            pltpu.SemaphoreType.DMA,
        ],
    )
    return f(x_user, x_movie, edge_label_index.reshape(2 * n_edges))


# speedup vs baseline: 1.1974x; 1.1354x over previous
import jax
import jax.numpy as jnp
from jax import lax
from jax.experimental import pallas as pl
from jax.experimental.pallas import tpu as pltpu
from jax.experimental.pallas import tpu_sc as plsc

D = 128
L = 16
NC = 2
NS = 16
NW = NC * NS
B = 80
NG = B // L


def _ip_body(xu, xm, eidx, out, idxu_all, idxm_all,
             u0, m0, u1, m1, ob0, ob1, tmat,
             su0, sm0, su1, sm1, so0, so1):
    wid = lax.axis_index("s") * NC + lax.axis_index("c")
    n_edges = out.shape[0]
    epw = n_edges // NW
    chunks = epw // B
    base = wid * epw

    pltpu.sync_copy(eidx.at[pl.ds(base, epw)], idxu_all)
    pltpu.sync_copy(eidx.at[pl.ds(n_edges + base, epw)], idxm_all)

    def issue(c, ub, mb, su, sm):
        o = c * B
        pltpu.async_copy(xu.at[idxu_all.at[pl.ds(o, B)]], ub, su)
        pltpu.async_copy(xm.at[idxm_all.at[pl.ds(o, B)]], mb, sm)

    def wait(ub, mb, su, sm):
        pltpu.make_async_copy(xu.at[idxu_all.at[pl.ds(0, B)]], ub, su).wait()
        pltpu.make_async_copy(xm.at[idxm_all.at[pl.ds(0, B)]], mb, sm).wait()

    lane = lax.iota(jnp.int32, L)

    def rowsum(gi, ob):
        rows = [tmat[r, pl.ds(gi * 17, L)] for r in range(L)]
        while len(rows) > 1:
            rows = [rows[i] + rows[i + 1] for i in range(0, len(rows), 2)]
        ob[pl.ds(gi * L, L)] = rows[0]

    def compute(c, ub, mb, ob, so):
        @pl.when(c >= 2)
        def _():
            pltpu.make_async_copy(ob, out.at[pl.ds(0, B)], so).wait()

        def group_body(g, c2):
            e0 = g * L
            ub_g = ub.at[pl.ds(e0, L)]
            mb_g = mb.at[pl.ds(e0, L)]
            colv = lane * 0 + g * 17
            W = 2 * L
            for t in range(L):
                a0 = ub_g[t, pl.ds(0, W)] * mb_g[t, pl.ds(0, W)]
                a1 = ub_g[t, pl.ds(W, W)] * mb_g[t, pl.ds(W, W)]
                a0 = a0 + ub_g[t, pl.ds(2 * W, W)] * mb_g[t, pl.ds(2 * W, W)]
                a1 = a1 + ub_g[t, pl.ds(3 * W, W)] * mb_g[t, pl.ds(3 * W, W)]
                f0, f1 = plsc.unpack(a0 + a1, format=plsc.PackFormat.INTERLEAVED)
                plsc.store_scatter(tmat, [lane, colv + t], f0 + f1)
            return c2

        lax.fori_loop(0, NG, group_body, 0)
        for g in range(NG):
            rowsum(g, ob)
        pltpu.async_copy(ob, out.at[pl.ds(base + c * B, B)], so)

    issue(0, u0, m0, su0, sm0)

    def pair_body(j, carry):
        c = 2 * j
        issue(c + 1, u1, m1, su1, sm1)
        wait(u0, m0, su0, sm0)
        compute(c, u0, m0, ob0, so0)
        issue(c + 2, u0, m0, su0, sm0)
        wait(u1, m1, su1, sm1)
        compute(c + 1, u1, m1, ob1, so1)
        return carry

    lax.fori_loop(0, (chunks - 1) // 2, pair_body, 0)
    wait(u0, m0, su0, sm0)
    compute(chunks - 1, u0, m0, ob0, so0)

    pltpu.make_async_copy(ob0, out.at[pl.ds(0, B)], so0).wait()
    pltpu.make_async_copy(ob1, out.at[pl.ds(0, B)], so1).wait()


def kernel(x_user, x_movie, edge_label_index):
    n_edges = edge_label_index.shape[1]
    epw = n_edges // NW
    mesh = plsc.VectorSubcoreMesh(core_axis_name="c", subcore_axis_name="s")
    f = pl.kernel(
        _ip_body,
        out_type=jax.ShapeDtypeStruct((n_edges,), jnp.float32),
        mesh=mesh,
        compiler_params=pltpu.CompilerParams(needs_layout_passes=False,
                                             use_tc_tiling_on_sc=False),
        scratch_types=[
            pltpu.VMEM((epw,), jnp.int32),
            pltpu.VMEM((epw,), jnp.int32),
            pltpu.VMEM((B, D), jnp.bfloat16),
            pltpu.VMEM((B, D), jnp.bfloat16),
            pltpu.VMEM((B, D), jnp.bfloat16),
            pltpu.VMEM((B, D), jnp.bfloat16),
            pltpu.VMEM((B,), jnp.float32),
            pltpu.VMEM((B,), jnp.float32),
            pltpu.VMEM((L, 85), jnp.float32),
            pltpu.SemaphoreType.DMA,
            pltpu.SemaphoreType.DMA,
            pltpu.SemaphoreType.DMA,
            pltpu.SemaphoreType.DMA,
            pltpu.SemaphoreType.DMA,
            pltpu.SemaphoreType.DMA,
        ],
    )
    return f(x_user.astype(jnp.bfloat16), x_movie.astype(jnp.bfloat16),
             edge_label_index.reshape(2 * n_edges))


# speedup vs baseline: 1.4612x; 1.2203x over previous
import jax
import jax.numpy as jnp
from jax import lax
from jax.experimental import pallas as pl
from jax.experimental.pallas import tpu as pltpu
from jax.experimental.pallas import tpu_sc as plsc

D = 128
L = 16
NC = 2
NS = 16
NW = NC * NS
B = 80
NG = B // L


def _ip_body(xu, xm, eidx, out, idxu_all, idxm_all,
             u0, m0, u1, m1, ob0, ob1, tmat,
             su0, sm0, su1, sm1, so0, so1):
    wid = lax.axis_index("s") * NC + lax.axis_index("c")
    n_edges = out.shape[0]
    epw = n_edges // NW
    chunks = epw // B
    base = wid * epw

    pltpu.sync_copy(eidx.at[pl.ds(base, epw)], idxu_all)
    pltpu.sync_copy(eidx.at[pl.ds(n_edges + base, epw)], idxm_all)

    def issue(c, ub, mb, su, sm):
        o = c * B
        pltpu.async_copy(xu.at[idxu_all.at[pl.ds(o, B)]], ub, su)
        pltpu.async_copy(xm.at[idxm_all.at[pl.ds(o, B)]], mb, sm)

    def wait(ub, mb, su, sm):
        pltpu.make_async_copy(xu.at[idxu_all.at[pl.ds(0, B)]], ub, su).wait()
        pltpu.make_async_copy(xm.at[idxm_all.at[pl.ds(0, B)]], mb, sm).wait()

    lane = lax.iota(jnp.int32, L)

    def rowsum(gi, ob):
        rows = [tmat[r, pl.ds(gi * 17, L)] for r in range(L)]
        while len(rows) > 1:
            rows = [rows[i] + rows[i + 1] for i in range(0, len(rows), 2)]
        ob[pl.ds(gi * L, L)] = rows[0]

    def compute(c, ub, mb, ob, so):
        @pl.when(c >= 2)
        def _():
            pltpu.make_async_copy(ob, out.at[pl.ds(0, B)], so).wait()

        def group_body(g, c2):
            e0 = g * L
            ub_g = ub.at[pl.ds(e0, L)]
            mb_g = mb.at[pl.ds(e0, L)]
            colv = lane * 0 + g * 17
            W = 2 * L
            fs = []
            for t in range(L):
                a0 = ub_g[t, pl.ds(0, W)] * mb_g[t, pl.ds(0, W)]
                a1 = ub_g[t, pl.ds(W, W)] * mb_g[t, pl.ds(W, W)]
                a0 = a0 + ub_g[t, pl.ds(2 * W, W)] * mb_g[t, pl.ds(2 * W, W)]
                a1 = a1 + ub_g[t, pl.ds(3 * W, W)] * mb_g[t, pl.ds(3 * W, W)]
                f0, f1 = plsc.unpack(a0 + a1, format=plsc.PackFormat.INTERLEAVED)
                fs.append(f0 + f1)
            for t in range(L):
                plsc.store_scatter(tmat, [lane, colv + t], fs[t])
            return c2

        lax.fori_loop(0, NG, group_body, 0)
        for g in range(NG):
            rowsum(g, ob)
        pltpu.async_copy(ob, out.at[pl.ds(base + c * B, B)], so)

    issue(0, u0, m0, su0, sm0)

    def pair_body(j, carry):
        c = 2 * j
        issue(c + 1, u1, m1, su1, sm1)
        wait(u0, m0, su0, sm0)
        compute(c, u0, m0, ob0, so0)
        issue(c + 2, u0, m0, su0, sm0)
        wait(u1, m1, su1, sm1)
        compute(c + 1, u1, m1, ob1, so1)
        return carry

    lax.fori_loop(0, (chunks - 1) // 2, pair_body, 0)
    wait(u0, m0, su0, sm0)
    compute(chunks - 1, u0, m0, ob0, so0)

    pltpu.make_async_copy(ob0, out.at[pl.ds(0, B)], so0).wait()
    pltpu.make_async_copy(ob1, out.at[pl.ds(0, B)], so1).wait()


def kernel(x_user, x_movie, edge_label_index):
    n_edges = edge_label_index.shape[1]
    epw = n_edges // NW
    mesh = plsc.VectorSubcoreMesh(core_axis_name="c", subcore_axis_name="s")
    f = pl.kernel(
        _ip_body,
        out_type=jax.ShapeDtypeStruct((n_edges,), jnp.float32),
        mesh=mesh,
        compiler_params=pltpu.CompilerParams(needs_layout_passes=False,
                                             use_tc_tiling_on_sc=False),
        scratch_types=[
            pltpu.VMEM((epw,), jnp.int32),
            pltpu.VMEM((epw,), jnp.int32),
            pltpu.VMEM((B, D), jnp.bfloat16),
            pltpu.VMEM((B, D), jnp.bfloat16),
            pltpu.VMEM((B, D), jnp.bfloat16),
            pltpu.VMEM((B, D), jnp.bfloat16),
            pltpu.VMEM((B,), jnp.float32),
            pltpu.VMEM((B,), jnp.float32),
            pltpu.VMEM((L, 85), jnp.float32),
            pltpu.SemaphoreType.DMA,
            pltpu.SemaphoreType.DMA,
            pltpu.SemaphoreType.DMA,
            pltpu.SemaphoreType.DMA,
            pltpu.SemaphoreType.DMA,
            pltpu.SemaphoreType.DMA,
        ],
    )
    return f(x_user.astype(jnp.bfloat16), x_movie.astype(jnp.bfloat16),
             edge_label_index.reshape(2 * n_edges))
